# Initial kernel scaffold; baseline (speedup 1.0000x reference)
import functools

import jax
import jax.numpy as jnp
from jax import lax
from jax.experimental import pallas as pl
from jax.experimental.pallas import tpu as pltpu
from jax.experimental.pallas import tpu_sc as plsc

N = 4096
K = 16
D = 256
RT = 512
RM = 256
GCH = 128


def _topk_body(x1_ref, x2t_ref, idx_ref):
    x1 = x1_ref[...]
    x2t = x2t_ref[...]
    dot = jnp.dot(x1, x2t, preferred_element_type=jnp.float32)
    n1 = jnp.sum(x1 * x1, axis=1, keepdims=True)
    n2 = jnp.sum(x2t * x2t, axis=0, keepdims=True)
    d = (-2.0 * dot + n1) + n2
    d = jnp.maximum(d, 0.0)
    iota = lax.broadcasted_iota(jnp.int32, (1, N), 1)
    cols = []
    for _ in range(K):
        m = jnp.min(d, axis=1, keepdims=True)
        ca = jnp.where(d <= m, iota, N)
        a = jnp.min(ca, axis=1, keepdims=True)
        cols.append(a)
        d = jnp.where(ca == a, 3.4e38, d)
    idx_ref[...] = jnp.concatenate(cols, axis=1)


def _t2_body(p2_ref, w0aT_ref, t2_ref):
    t2_ref[...] = jnp.dot(p2_ref[...], w0aT_ref[...],
                          preferred_element_type=jnp.float32)


def _gather_sc(t2, xyzt, idxf):
    nw = 32
    per_w = idxf.shape[0] // nw
    nch = per_w // GCH

    mesh = plsc.VectorSubcoreMesh(core_axis_name="c", subcore_axis_name="s")

    @functools.partial(
        pl.kernel, mesh=mesh,
        out_type=(jax.ShapeDtypeStruct((N * K, D), jnp.float32),
                  jax.ShapeDtypeStruct((N * K, 16), jnp.float32)),
        scratch_types=[pltpu.VMEM((GCH,), jnp.int32),
                       pltpu.VMEM((GCH, D), jnp.float32),
                       pltpu.VMEM((GCH, 16), jnp.float32),
                       pltpu.SemaphoreType.DMA,
                       pltpu.SemaphoreType.DMA],
    )
    def k(t2_hbm, xyz_hbm, idx_hbm, of_hbm, ox_hbm, idx_v, f_v, x_v, s1, s2):
        wid = lax.axis_index("s") * 2 + lax.axis_index("c")
        base = wid * per_w

        @pl.loop(0, nch)
        def _(ci):
            off = base + ci * GCH
            pltpu.sync_copy(idx_hbm.at[pl.ds(off, GCH)], idx_v)
            cf = pltpu.async_copy(t2_hbm.at[idx_v], f_v, s1)
            cx = pltpu.async_copy(xyz_hbm.at[idx_v], x_v, s2)
            cf.wait()
            cx.wait()
            pltpu.sync_copy(f_v, of_hbm.at[pl.ds(off, GCH)])
            pltpu.sync_copy(x_v, ox_hbm.at[pl.ds(off, GCH)])

    return k(t2, xyzt, idxf)


def _mlp_body(gf_ref, gx_ref, x1_ref, p1_ref, w0bT_ref, w0cT_ref, b0_ref,
              w1T_ref, b1_ref, wn0T_ref, bn0_ref, wn1T_ref, bn1_ref,
              wn2T_ref, bn2_ref, out_ref):
    gf = gf_ref[...]
    gx = gx_ref[...]
    x1 = x1_ref[...]
    dirp = gx[:, :, 0:8] - x1[:, None, :]
    dir2 = dirp.reshape(RM * K, 8)
    t1 = jnp.dot(p1_ref[...], w0bT_ref[...],
                 preferred_element_type=jnp.float32)
    dw = jnp.dot(dir2, w0cT_ref[...],
                 preferred_element_type=jnp.float32)
    z0 = gf + t1[:, None, :] + dw.reshape(RM, K, D) + b0_ref[...][None]
    a0 = jnp.where(z0 >= 0, z0, 0.1 * z0)
    z1 = jnp.dot(a0.reshape(RM * K, D), w1T_ref[...],
                 preferred_element_type=jnp.float32) + b1_ref[...]
    a1 = jnp.where(z1 >= 0, z1, 0.1 * z1)
    w = jnp.maximum(jnp.dot(dir2, wn0T_ref[...],
                            preferred_element_type=jnp.float32)
                    + bn0_ref[...], 0.0)
    w = jnp.maximum(jnp.dot(w, wn1T_ref[...],
                            preferred_element_type=jnp.float32)
                    + bn1_ref[...], 0.0)
    w = jnp.maximum(jnp.dot(w, wn2T_ref[...],
                            preferred_element_type=jnp.float32)
                    + bn2_ref[...], 0.0)
    prod = (a1 * w).reshape(RM, K, D)
    out_ref[...] = jnp.sum(prod, axis=1) * (1.0 / K)


def _full(shape):
    return pl.BlockSpec(shape, lambda i: tuple(0 for _ in shape))


def kernel(xyz1, xyz2, points1, points2, xyz1_, xyz2_, W0, b0, W1, b1,
           Wn0, bn0, Wn1, bn1, Wn2, bn2):
    x1 = xyz1[0]
    x2 = xyz2[0]
    p1 = points1[0]
    p2 = points2[0]
    x1p = jnp.pad(x1, ((0, 0), (0, 5)))
    x2pT = jnp.pad(x2, ((0, 0), (0, 5))).T
    xyzt = jnp.pad(x2, ((0, 0), (0, 13)))

    W0aT = W0[:, 0:D].T
    W0bT = W0[:, D:2 * D].T
    W0cT = jnp.pad(W0[:, 2 * D:2 * D + 3].T, ((0, 5), (0, 0)))
    Wn0T = jnp.pad(Wn0.T, ((0, 5), (0, 0)))
    Wn1T = Wn1.T
    Wn2T = Wn2.T
    b0r = b0[None, :]
    b1r = b1[None, :]
    bn0r = bn0[None, :]
    bn1r = bn1[None, :]
    bn2r = bn2[None, :]

    idx = pl.pallas_call(
        _topk_body,
        grid=(N // RT,),
        in_specs=[pl.BlockSpec((RT, 8), lambda i: (i, 0)),
                  pl.BlockSpec((8, N), lambda i: (0, 0))],
        out_specs=pl.BlockSpec((RT, K), lambda i: (i, 0)),
        out_shape=jax.ShapeDtypeStruct((N, K), jnp.int32),
    )(x1p, x2pT)

    t2 = pl.pallas_call(
        _t2_body,
        out_shape=jax.ShapeDtypeStruct((N, D), jnp.float32),
    )(p2, W0aT)

    gf, gx = _gather_sc(t2, xyzt, idx.reshape(N * K))
    gf3 = gf.reshape(N, K, D)
    gx3 = gx.reshape(N, K, 16)

    out = pl.pallas_call(
        _mlp_body,
        grid=(N // RM,),
        in_specs=[pl.BlockSpec((RM, K, D), lambda i: (i, 0, 0)),
                  pl.BlockSpec((RM, K, 16), lambda i: (i, 0, 0)),
                  pl.BlockSpec((RM, 8), lambda i: (i, 0)),
                  pl.BlockSpec((RM, D), lambda i: (i, 0)),
                  _full((D, D)),
                  _full((8, D)),
                  _full((1, D)),
                  _full((D, D)),
                  _full((1, D)),
                  _full((8, 8)),
                  _full((1, 8)),
                  _full((8, 8)),
                  _full((1, 8)),
                  _full((8, D)),
                  _full((1, D))],
        out_specs=pl.BlockSpec((RM, D), lambda i: (i, 0)),
        out_shape=jax.ShapeDtypeStruct((N, D), jnp.float32),
    )(gf3, gx3, x1p, p1, W0bT, W0cT, b0r, W1T, b1r,
      Wn0T, bn0r, Wn1T, bn1r, Wn2T, bn2r)
    return out[None]


# speedup vs baseline: 9.5145x; 9.5145x over previous
import functools

import jax
import jax.numpy as jnp
from jax import lax
from jax.experimental import pallas as pl
from jax.experimental.pallas import tpu as pltpu
from jax.experimental.pallas import tpu_sc as plsc

N = 4096
K = 16
D = 256
RT = 512
RM = 256
GCH = 128


def _topk_body(x1_ref, x2t_ref, idx_ref):
    x1 = x1_ref[...]
    x2t = x2t_ref[...]
    dot = jnp.dot(x1, x2t, preferred_element_type=jnp.float32)
    n1 = jnp.sum(x1 * x1, axis=1, keepdims=True)
    n2 = jnp.sum(x2t * x2t, axis=0, keepdims=True)
    d = (-2.0 * dot + n1) + n2
    d = jnp.maximum(d, 0.0)
    iota = lax.broadcasted_iota(jnp.int32, (1, N), 1)
    cols = []
    for _ in range(K):
        m = jnp.min(d, axis=1, keepdims=True)
        ca = jnp.where(d <= m, iota, N)
        a = jnp.min(ca, axis=1, keepdims=True)
        cols.append(a)
        d = jnp.where(ca == a, 3.4e38, d)
    idx_ref[...] = jnp.concatenate(cols, axis=1)


DT = 384


def _t2_body(p2_ref, w0aT_ref, xyz_ref, t2_ref):
    t2 = jnp.dot(p2_ref[...], w0aT_ref[...],
                 preferred_element_type=jnp.float32)
    pad = jnp.zeros((t2.shape[0], DT - D - 16), jnp.float32)
    t2_ref[...] = jnp.concatenate([t2, xyz_ref[...], pad], axis=1)


def _gather_sc(tbl, idxf):
    nw = 32
    per_w = idxf.shape[0] // nw
    nch = per_w // GCH

    mesh = plsc.VectorSubcoreMesh(core_axis_name="c", subcore_axis_name="s")

    @functools.partial(
        pl.kernel, mesh=mesh,
        out_type=jax.ShapeDtypeStruct((N * K, DT), jnp.float32),
        scratch_types=[pltpu.VMEM((GCH,), jnp.int32),
                       pltpu.VMEM((GCH, DT), jnp.float32),
                       pltpu.SemaphoreType.DMA],
    )
    def k(tbl_hbm, idx_hbm, of_hbm, idx_v, f_v, s1):
        wid = lax.axis_index("s") * 2 + lax.axis_index("c")
        base = wid * per_w

        @pl.loop(0, nch)
        def _(ci):
            off = base + ci * GCH
            pltpu.sync_copy(idx_hbm.at[pl.ds(off, GCH)], idx_v)
            pltpu.async_copy(tbl_hbm.at[idx_v], f_v, s1).wait()
            pltpu.sync_copy(f_v, of_hbm.at[pl.ds(off, GCH)])

    return k(tbl, idxf)


def _mlp_body(g_ref, x1_ref, p1_ref, w0bT_ref, w0cT_ref, b0_ref,
              w1T_ref, b1_ref, wn0T_ref, bn0_ref, wn1T_ref, bn1_ref,
              wn2T_ref, bn2_ref, out_ref):
    gf = g_ref[:, :, 0:D]
    gx = g_ref[:, :, D:D + 8]
    x1 = x1_ref[...]
    dirp = gx - x1[:, None, :]
    dir2 = dirp.reshape(RM * K, 8)
    t1 = jnp.dot(p1_ref[...], w0bT_ref[...],
                 preferred_element_type=jnp.float32)
    dw = jnp.dot(dir2, w0cT_ref[...],
                 preferred_element_type=jnp.float32)
    z0 = gf + t1[:, None, :] + dw.reshape(RM, K, D) + b0_ref[...][None]
    a0 = jnp.where(z0 >= 0, z0, 0.1 * z0)
    z1 = jnp.dot(a0.reshape(RM * K, D), w1T_ref[...],
                 preferred_element_type=jnp.float32) + b1_ref[...]
    a1 = jnp.where(z1 >= 0, z1, 0.1 * z1)
    w = jnp.maximum(jnp.dot(dir2, wn0T_ref[...],
                            preferred_element_type=jnp.float32)
                    + bn0_ref[...], 0.0)
    w = jnp.maximum(jnp.dot(w, wn1T_ref[...],
                            preferred_element_type=jnp.float32)
                    + bn1_ref[...], 0.0)
    w = jnp.maximum(jnp.dot(w, wn2T_ref[...],
                            preferred_element_type=jnp.float32)
                    + bn2_ref[...], 0.0)
    prod = (a1 * w).reshape(RM, K, D)
    out_ref[...] = jnp.sum(prod, axis=1) * (1.0 / K)


def _full(shape):
    return pl.BlockSpec(shape, lambda i: tuple(0 for _ in shape))


def kernel(xyz1, xyz2, points1, points2, xyz1_, xyz2_, W0, b0, W1, b1,
           Wn0, bn0, Wn1, bn1, Wn2, bn2):
    x1 = xyz1[0]
    x2 = xyz2[0]
    p1 = points1[0]
    p2 = points2[0]
    x1p = jnp.pad(x1, ((0, 0), (0, 5)))
    x2pT = jnp.pad(x2, ((0, 0), (0, 5))).T
    xyzt = jnp.pad(x2, ((0, 0), (0, 13)))

    W0aT = W0[:, 0:D].T
    W0bT = W0[:, D:2 * D].T
    W0cT = jnp.pad(W0[:, 2 * D:2 * D + 3].T, ((0, 5), (0, 0)))
    W1T = W1.T
    Wn0T = jnp.pad(Wn0.T, ((0, 5), (0, 0)))
    Wn1T = Wn1.T
    Wn2T = Wn2.T
    b0r = b0[None, :]
    b1r = b1[None, :]
    bn0r = bn0[None, :]
    bn1r = bn1[None, :]
    bn2r = bn2[None, :]

    idx = pl.pallas_call(
        _topk_body,
        grid=(N // RT,),
        in_specs=[pl.BlockSpec((RT, 8), lambda i: (i, 0)),
                  pl.BlockSpec((8, N), lambda i: (0, 0))],
        out_specs=pl.BlockSpec((RT, K), lambda i: (i, 0)),
        out_shape=jax.ShapeDtypeStruct((N, K), jnp.int32),
    )(x1p, x2pT)

    tbl = pl.pallas_call(
        _t2_body,
        out_shape=jax.ShapeDtypeStruct((N, DT), jnp.float32),
    )(p2, W0aT, xyzt)

    g = _gather_sc(tbl, idx.reshape(N * K))
    g3 = g.reshape(N, K, DT)

    out = pl.pallas_call(
        _mlp_body,
        grid=(N // RM,),
        in_specs=[pl.BlockSpec((RM, K, DT), lambda i: (i, 0, 0)),
                  pl.BlockSpec((RM, 8), lambda i: (i, 0)),
                  pl.BlockSpec((RM, D), lambda i: (i, 0)),
                  _full((D, D)),
                  _full((8, D)),
                  _full((1, D)),
                  _full((D, D)),
                  _full((1, D)),
                  _full((8, 8)),
                  _full((1, 8)),
                  _full((8, 8)),
                  _full((1, 8)),
                  _full((8, D)),
                  _full((1, D))],
        out_specs=pl.BlockSpec((RM, D), lambda i: (i, 0)),
        out_shape=jax.ShapeDtypeStruct((N, D), jnp.float32),
    )(g3, x1p, p1, W0bT, W0cT, b0r, W1T, b1r,
      Wn0T, bn0r, Wn1T, bn1r, Wn2T, bn2r)
    return out[None]
